# baseline (speedup 1.0000x reference)
import functools

import jax
import jax.numpy as jnp
from jax import lax
from jax.experimental import pallas as pl
from jax.experimental.pallas import tpu as pltpu
from jax.experimental.pallas import tpu_sc as plsc

N = 10000
E = 160000
D = 256
H = 512

NC = 2
NS = 16
L = 16

CHUNK = 1256
N_PAD = 8 * CHUNK
NPASS = 4
ACC_ROWS = CHUNK + 24
TRASH = CHUNK
EPT = E // NS
SB = 2000


def _sc_edge_body(h_hbm, src_hbm, dst_hbm, ea_hbm, tvec_hbm, agg_hbm,
                  sd, ss, ce_a, ce_b,
                  hb0, hb1, hb2, hb3, eb0, eb1, eb2, eb3,
                  ob0, ob1, wbuf, abuf, tvec_v, ix0, ix1,
                  acc, semh0, semh1, semh2, semh3,
                  seme0, seme1, seme2, seme3, sems0, sems1):
    c = lax.axis_index("c")
    s = lax.axis_index("s")

    pltpu.sync_copy(tvec_hbm, tvec_v)
    tv = tvec_v[...]

    ebase = s * EPT
    HB = (hb0, hb1, hb2, hb3)
    EB = (eb0, eb1, eb2, eb3)
    OB = (ob0, ob1)
    IX = (ix0, ix1)
    SEMH = (semh0, semh1, semh2, semh3)
    SEME = (seme0, seme1, seme2, seme3)
    SEMS = (sems0, sems1)

    def _issue(j, b):
        pk = ce_a[pl.ds(j * L, L)]
        isrc = ce_b[pl.ds(j * L, L)]
        pltpu.async_copy(h_hbm.at[isrc], HB[b], SEMH[b])
        pltpu.async_copy(ea_hbm.at[pk & 0x3FFFF], EB[b], SEME[b])

    def _wait(b):
        pltpu.make_async_copy(h_hbm.at[pl.ds(0, L)], HB[b], SEMH[b]).wait()
        pltpu.make_async_copy(ea_hbm.at[pl.ds(0, L)], EB[b], SEME[b]).wait()

    def _drain(b):
        for q in range(4):
            pltpu.make_async_copy(
                h_hbm.at[pl.ds(0, L), pl.ds(0, 128)], OB[b].at[q],
                SEMS[b]).wait()

    def _process(j, g, o, tvv):
        @pl.when(j >= 2)
        def _():
            _drain(o)

        pk = ce_a[pl.ds(j * L, L)]
        idst4 = (pk >> 18) * 4
        for q in range(4):
            IX[o][q, :] = idst4 + q
        hb, eb, ob = HB[g], EB[g], OB[o]

        def _edge(e, _):
            for g in range(2):
                base = g * 8
                hs = [hb[e, pl.ds((base + i) * L, L)] for i in range(8)]
                es = [eb[e, pl.ds((base + i) * L, L)] for i in range(8)]
                ms = [jnp.maximum(hs[i] + es[i], 0.0) + 1e-7
                      for i in range(8)]
                xs = [jnp.exp(ms[i] * tvv) for i in range(8)]
                ns = [xs[i] * ms[i] for i in range(8)]
                for i in range(8):
                    ob[g, e, pl.ds(i * L, L)] = ns[i]
                for i in range(8):
                    ob[2 + g, e, pl.ds(i * L, L)] = xs[i]
            return 0
        lax.fori_loop(0, L, _edge, 0)

        for q in range(4):
            pltpu.async_copy(ob.at[q], acc.at[IX[o].at[q]], SEMS[o], add=True)

    for p in range(NPASS):
        lo = (2 * p + c) * CHUNK

        def _zfill(k, _):
            i = k // 8
            j = (k % 8) * L
            wbuf[i, pl.ds(j, L)] = jnp.zeros((L,), jnp.float32)
            return 0
        lax.fori_loop(0, 32 * 8, _zfill, 0)

        def _zero(i, _):
            pltpu.sync_copy(
                wbuf, acc.at[pl.ds((s * (ACC_ROWS // NS) + i * 8) * 4, 32)])
            return 0
        lax.fori_loop(0, ACC_ROWS // NS // 8, _zero, 0)
        plsc.subcore_barrier()

        def _strip(cb, _):
            pltpu.sync_copy(dst_hbm.at[pl.ds(ebase + cb * SB, SB)], sd)
            pltpu.sync_copy(src_hbm.at[pl.ds(ebase + cb * SB, SB)], ss)

            def _compact(i, cnt):
                off = i * L
                d = sd[pl.ds(off, L)]
                dl = d - lo
                m = (dl >= 0) & (dl < CHUNK)
                mi = m.astype(jnp.int32)
                pos = cnt + plsc.cumsum(mi) - 1
                eidv = lax.iota(jnp.int32, L) + (ebase + cb * SB + off)
                plsc.store_scatter(ce_a, [pos], eidv | (dl << 18), mask=m)
                plsc.store_scatter(ce_b, [pos], ss[pl.ds(off, L)], mask=m)
                return cnt + jnp.sum(mi)
            cnt = lax.fori_loop(0, SB // L, _compact, jnp.int32(0))

            ce_a[pl.ds(cnt, L)] = jnp.full((L,), TRASH << 18, jnp.int32)
            ce_b[pl.ds(cnt, L)] = jnp.zeros((L,), jnp.int32)

            nblk = (cnt + L - 1) // L

            for pre in range(3):
                @pl.when(pre < nblk)
                def _():
                    _issue(pre, pre)

            def _quad(i4, _):
                jb = i4 * 4
                for g in range(4):
                    j = jb + g

                    @pl.when(j < nblk)
                    def _():
                        @pl.when(j + 3 < nblk)
                        def _():
                            _issue(j + 3, (g + 3) % 4)
                        _wait(g)
                        _process(j, g, g % 2, tv)
                return 0
            lax.fori_loop(0, (nblk + 3) // 4, _quad, 0)

            for b in range(2):
                @pl.when((nblk >= 1) & ((nblk - 1) % 2 == b))
                def _():
                    _drain(b)

                @pl.when((nblk >= 2) & ((nblk - 2) % 2 == b))
                def _():
                    _drain(b)
            return 0
        lax.fori_loop(0, EPT // SB, _strip, 0)
        plsc.subcore_barrier()

        def _wout(i, _):
            blk = s + NS * i

            @pl.when(blk < CHUNK // 8)
            def _():
                r = blk * 8
                pltpu.sync_copy(acc.at[pl.ds(r * 4, 32)], wbuf)

                def _div(k, _):
                    row = k // (D // L)
                    f = k % (D // L)
                    q = f // 8
                    col = (f % 8) * L
                    nm = wbuf[row * 4 + q, pl.ds(col, L)]
                    dn = wbuf[row * 4 + 2 + q, pl.ds(col, L)]
                    abuf[row, pl.ds(f * L, L)] = nm / (dn + 1e-16)
                    return 0
                lax.fori_loop(0, 8 * D // L, _div, 0)
                pltpu.sync_copy(abuf, agg_hbm.at[pl.ds(lo + r, 8)])
            return 0
        lax.fori_loop(0, (CHUNK // 8 + NS - 1) // NS, _wout, 0)
        plsc.subcore_barrier()


@jax.jit
def _sc_edge(h, src, dst, ea, tvec):
    mesh = plsc.VectorSubcoreMesh(core_axis_name="c", subcore_axis_name="s")
    return pl.kernel(
        _sc_edge_body,
        out_type=jax.ShapeDtypeStruct((N_PAD, D), jnp.float32),
        mesh=mesh,
        compiler_params=pltpu.CompilerParams(needs_layout_passes=False),
        scratch_types=[
            pltpu.VMEM((SB,), jnp.int32),
            pltpu.VMEM((SB,), jnp.int32),
            pltpu.VMEM((SB + L,), jnp.int32),
            pltpu.VMEM((SB + L,), jnp.int32),
            pltpu.VMEM((L, D), jnp.float32),
            pltpu.VMEM((L, D), jnp.float32),
            pltpu.VMEM((L, D), jnp.float32),
            pltpu.VMEM((L, D), jnp.float32),
            pltpu.VMEM((L, D), jnp.float32),
            pltpu.VMEM((L, D), jnp.float32),
            pltpu.VMEM((L, D), jnp.float32),
            pltpu.VMEM((L, D), jnp.float32),
            pltpu.VMEM((4, L, 128), jnp.float32),
            pltpu.VMEM((4, L, 128), jnp.float32),
            pltpu.VMEM((32, 128), jnp.float32),
            pltpu.VMEM((8, D), jnp.float32),
            pltpu.VMEM((L,), jnp.float32),
            pltpu.VMEM((4, L), jnp.int32),
            pltpu.VMEM((4, L), jnp.int32),
            pltpu.VMEM_SHARED((ACC_ROWS * 4, 128), jnp.float32),
            pltpu.SemaphoreType.DMA,
            pltpu.SemaphoreType.DMA,
            pltpu.SemaphoreType.DMA,
            pltpu.SemaphoreType.DMA,
            pltpu.SemaphoreType.DMA,
            pltpu.SemaphoreType.DMA,
            pltpu.SemaphoreType.DMA,
            pltpu.SemaphoreType.DMA,
            pltpu.SemaphoreType.DMA,
            pltpu.SemaphoreType.DMA,
        ],
    )(h, src, dst, ea, tvec)


def _pre_body(x_ref, g_ref, b_ref, h_ref):
    xv = x_ref[...]
    mu = jnp.mean(xv, axis=-1, keepdims=True)
    var = jnp.mean(jnp.square(xv - mu), axis=-1, keepdims=True)
    hv = (xv - mu) * lax.rsqrt(var + 1e-5) * g_ref[...] + b_ref[...]
    h_ref[...] = jnp.maximum(hv, 0.0)


@jax.jit
def _pre(x, g, b):
    blk = 1000
    return pl.pallas_call(
        _pre_body,
        grid=(N // blk,),
        in_specs=[
            pl.BlockSpec((blk, D), lambda i: (i, 0)),
            pl.BlockSpec((1, D), lambda i: (0, 0)),
            pl.BlockSpec((1, D), lambda i: (0, 0)),
        ],
        out_specs=pl.BlockSpec((blk, D), lambda i: (i, 0)),
        out_shape=jax.ShapeDtypeStruct((N, D), jnp.float32),
    )(x, g, b)


def _post_body(x_ref, h_ref, a_ref, W1_ref, b1_ref, g2_ref, bt2_ref,
               W2_ref, b2_ref, o_ref):
    z = a_ref[...] + h_ref[...]
    z = jnp.dot(z, W1_ref[...], preferred_element_type=jnp.float32) + b1_ref[...]
    mu = jnp.mean(z, axis=-1, keepdims=True)
    var = jnp.mean(jnp.square(z - mu), axis=-1, keepdims=True)
    z = (z - mu) * lax.rsqrt(var + 1e-5) * g2_ref[...] + bt2_ref[...]
    z = jnp.maximum(z, 0.0)
    o_ref[...] = (x_ref[...] + b2_ref[...]
                  + jnp.dot(z, W2_ref[...], preferred_element_type=jnp.float32))


@jax.jit
def _post(x, h, a, W1, b1, g2, bt2, W2, b2):
    blk = 1000
    return pl.pallas_call(
        _post_body,
        grid=(N // blk,),
        in_specs=[
            pl.BlockSpec((blk, D), lambda i: (i, 0)),
            pl.BlockSpec((blk, D), lambda i: (i, 0)),
            pl.BlockSpec((blk, D), lambda i: (i, 0)),
            pl.BlockSpec((D, H), lambda i: (0, 0)),
            pl.BlockSpec((1, H), lambda i: (0, 0)),
            pl.BlockSpec((1, H), lambda i: (0, 0)),
            pl.BlockSpec((1, H), lambda i: (0, 0)),
            pl.BlockSpec((H, D), lambda i: (0, 0)),
            pl.BlockSpec((1, D), lambda i: (0, 0)),
        ],
        out_specs=pl.BlockSpec((blk, D), lambda i: (i, 0)),
        out_shape=jax.ShapeDtypeStruct((N, D), jnp.float32),
    )(x, h, a, W1, b1, g2, bt2, W2, b2)


def kernel(x, edge_index, edge_attr, ln_g, ln_b, t, W1, b1, g2, bt2, W2, b2):
    src = edge_index[0]
    dst = edge_index[1]
    h = _pre(x, ln_g.reshape(1, D), ln_b.reshape(1, D))
    tvec = jnp.broadcast_to(jnp.asarray(t, jnp.float32).reshape(()), (L,))
    agg = _sc_edge(h, src, dst, edge_attr, tvec)[:N]
    return _post(x, h, agg, W1, b1.reshape(1, H), g2.reshape(1, H),
                 bt2.reshape(1, H), W2, b2.reshape(1, D))


# speedup vs baseline: 1.1807x; 1.1807x over previous
import functools

import jax
import jax.numpy as jnp
from jax import lax
from jax.experimental import pallas as pl
from jax.experimental.pallas import tpu as pltpu
from jax.experimental.pallas import tpu_sc as plsc

N = 10000
E = 160000
D = 256
H = 512

NC = 2
NS = 16
L = 16

CHUNK = 2512
N_PAD = 4 * CHUNK
ACC_ROWS = CHUNK + 48
TRASH = CHUNK
EPT = E // NS
SB = 2000


def _sc_edge_body(h_hbm, src_hbm, dst_hbm, ea_hbm, tvec_hbm, agg_hbm,
                  sd, ss, ce_a, ce_b,
                  hb0, hb1, eb0, eb1, ob0, ob1, wbuf, abuf, tvec_v, ix0, ix1,
                  acc, semh0, semh1, seme0, seme1, sems0, sems1):
    c = lax.axis_index("c")
    s = lax.axis_index("s")

    pltpu.sync_copy(tvec_hbm, tvec_v)
    tv = tvec_v[...]

    ebase = s * EPT
    HB = (hb0, hb1)
    EB = (eb0, eb1)
    OB = (ob0, ob1)
    IX = (ix0, ix1)
    SEMH = (semh0, semh1)
    SEME = (seme0, seme1)
    SEMS = (sems0, sems1)

    def _issue(j, b):
        pk = ce_a[pl.ds(j * L, L)]
        isrc = ce_b[pl.ds(j * L, L)]
        pltpu.async_copy(h_hbm.at[isrc], HB[b], SEMH[b])
        pltpu.async_copy(ea_hbm.at[pk & 0x3FFFF], EB[b], SEME[b])

    def _wait(b):
        pltpu.make_async_copy(h_hbm.at[pl.ds(0, L)], HB[b], SEMH[b]).wait()
        pltpu.make_async_copy(ea_hbm.at[pl.ds(0, L)], EB[b], SEME[b]).wait()

    def _drain(b):
        for q in range(4):
            pltpu.make_async_copy(
                h_hbm.at[pl.ds(0, L), pl.ds(0, 128)], OB[b].at[q],
                SEMS[b]).wait()

    def _process(j, b, tvv):
        @pl.when(j >= 2)
        def _():
            _drain(b)

        pk = ce_a[pl.ds(j * L, L)]
        idst4 = (pk >> 18) * 4
        for q in range(4):
            IX[b][q, :] = idst4 + q
        hb, eb, ob = HB[b], EB[b], OB[b]

        def _edge(e, _):
            for g in range(2):
                base = g * 8
                hs = [hb[e, pl.ds((base + i) * L, L)] for i in range(8)]
                es = [eb[e, pl.ds((base + i) * L, L)] for i in range(8)]
                ms = [jnp.maximum(hs[i] + es[i], 0.0) + 1e-7
                      for i in range(8)]
                xs = [jnp.exp(ms[i] * tvv) for i in range(8)]
                ns = [xs[i] * ms[i] for i in range(8)]
                for i in range(8):
                    ob[g, e, pl.ds(i * L, L)] = ns[i]
                for i in range(8):
                    ob[2 + g, e, pl.ds(i * L, L)] = xs[i]
            return 0
        lax.fori_loop(0, L, _edge, 0)

        for q in range(4):
            pltpu.async_copy(ob.at[q], acc.at[IX[b].at[q]], SEMS[b], add=True)

    for p in range(2):
        lo = (2 * p + c) * CHUNK

        def _zfill(k, _):
            i = k // 8
            j = (k % 8) * L
            wbuf[i, pl.ds(j, L)] = jnp.zeros((L,), jnp.float32)
            return 0
        lax.fori_loop(0, 32 * 8, _zfill, 0)

        def _zero(i, _):
            pltpu.async_copy(
                wbuf, acc.at[pl.ds((s * (ACC_ROWS // NS) + i * 8) * 4, 32)],
                semh0)
            return 0
        lax.fori_loop(0, ACC_ROWS // NS // 8, _zero, 0)

        def _zdrain(i, _):
            pltpu.make_async_copy(wbuf, acc.at[pl.ds(0, 32)], semh0).wait()
            return 0
        lax.fori_loop(0, ACC_ROWS // NS // 8, _zdrain, 0)
        plsc.subcore_barrier()

        def _strip(cb, _):
            cpd = pltpu.async_copy(
                dst_hbm.at[pl.ds(ebase + cb * SB, SB)], sd, semh0)
            cps = pltpu.async_copy(
                src_hbm.at[pl.ds(ebase + cb * SB, SB)], ss, seme0)
            cpd.wait()
            cps.wait()

            def _compact(i, cnt):
                off = i * L
                d = sd[pl.ds(off, L)]
                dl = d - lo
                m = (dl >= 0) & (dl < CHUNK)
                mi = m.astype(jnp.int32)
                pos = cnt + plsc.cumsum(mi) - 1
                eidv = lax.iota(jnp.int32, L) + (ebase + cb * SB + off)
                plsc.store_scatter(ce_a, [pos], eidv | (dl << 18), mask=m)
                plsc.store_scatter(ce_b, [pos], ss[pl.ds(off, L)], mask=m)
                return cnt + jnp.sum(mi)
            cnt = lax.fori_loop(0, SB // L, _compact, jnp.int32(0))

            ce_a[pl.ds(cnt, L)] = jnp.full((L,), TRASH << 18, jnp.int32)
            ce_b[pl.ds(cnt, L)] = jnp.zeros((L,), jnp.int32)

            nblk = (cnt + L - 1) // L

            @pl.when(nblk > 0)
            def _():
                _issue(0, 0)

            def _pair(i2, _):
                jb = i2 * 2
                for b in range(2):
                    j = jb + b

                    @pl.when(j < nblk)
                    def _():
                        @pl.when(j + 1 < nblk)
                        def _():
                            _issue(j + 1, 1 - b)
                        _wait(b)
                        _process(j, b, tv)
                return 0
            lax.fori_loop(0, (nblk + 1) // 2, _pair, 0)

            for b in range(2):
                @pl.when((nblk >= 1) & ((nblk - 1) % 2 == b))
                def _():
                    _drain(b)

                @pl.when((nblk >= 2) & ((nblk - 2) % 2 == b))
                def _():
                    _drain(b)
            return 0
        lax.fori_loop(0, EPT // SB, _strip, 0)
        plsc.subcore_barrier()

        def _wout(i, _):
            blk = s + NS * i

            @pl.when(blk < CHUNK // 8)
            def _():
                r = blk * 8
                pltpu.sync_copy(acc.at[pl.ds(r * 4, 32)], wbuf)

                @pl.when(i > 0)
                def _():
                    pltpu.make_async_copy(
                        abuf, agg_hbm.at[pl.ds(lo, 8)], sems0).wait()

                def _div(k, _):
                    row = k // (D // L)
                    f = k % (D // L)
                    q = f // 8
                    col = (f % 8) * L
                    nm = wbuf[row * 4 + q, pl.ds(col, L)]
                    dn = wbuf[row * 4 + 2 + q, pl.ds(col, L)]
                    abuf[row, pl.ds(f * L, L)] = nm / (dn + 1e-16)
                    return 0
                lax.fori_loop(0, 8 * D // L, _div, 0)
                pltpu.async_copy(abuf, agg_hbm.at[pl.ds(lo + r, 8)], sems0)
            return 0
        lax.fori_loop(0, (CHUNK // 8 + NS - 1) // NS, _wout, 0)
        pltpu.make_async_copy(abuf, agg_hbm.at[pl.ds(lo, 8)], sems0).wait()
        plsc.subcore_barrier()


@jax.jit
def _sc_edge(h, src, dst, ea, tvec):
    mesh = plsc.VectorSubcoreMesh(core_axis_name="c", subcore_axis_name="s")
    return pl.kernel(
        _sc_edge_body,
        out_type=jax.ShapeDtypeStruct((N_PAD, D), jnp.float32),
        mesh=mesh,
        compiler_params=pltpu.CompilerParams(needs_layout_passes=False),
        scratch_types=[
            pltpu.VMEM((SB,), jnp.int32),
            pltpu.VMEM((SB,), jnp.int32),
            pltpu.VMEM((SB + L,), jnp.int32),
            pltpu.VMEM((SB + L,), jnp.int32),
            pltpu.VMEM((L, D), jnp.float32),
            pltpu.VMEM((L, D), jnp.float32),
            pltpu.VMEM((L, D), jnp.float32),
            pltpu.VMEM((L, D), jnp.float32),
            pltpu.VMEM((4, L, 128), jnp.float32),
            pltpu.VMEM((4, L, 128), jnp.float32),
            pltpu.VMEM((32, 128), jnp.float32),
            pltpu.VMEM((8, D), jnp.float32),
            pltpu.VMEM((L,), jnp.float32),
            pltpu.VMEM((4, L), jnp.int32),
            pltpu.VMEM((4, L), jnp.int32),
            pltpu.VMEM_SHARED((ACC_ROWS * 4, 128), jnp.float32),
            pltpu.SemaphoreType.DMA,
            pltpu.SemaphoreType.DMA,
            pltpu.SemaphoreType.DMA,
            pltpu.SemaphoreType.DMA,
            pltpu.SemaphoreType.DMA,
            pltpu.SemaphoreType.DMA,
        ],
    )(h, src, dst, ea, tvec)


def _pre_body(x_ref, g_ref, b_ref, h_ref):
    xv = x_ref[...]
    mu = jnp.mean(xv, axis=-1, keepdims=True)
    var = jnp.mean(jnp.square(xv - mu), axis=-1, keepdims=True)
    hv = (xv - mu) * lax.rsqrt(var + 1e-5) * g_ref[...] + b_ref[...]
    h_ref[...] = jnp.maximum(hv, 0.0)


@jax.jit
def _pre(x, g, b):
    blk = 1000
    return pl.pallas_call(
        _pre_body,
        grid=(N // blk,),
        in_specs=[
            pl.BlockSpec((blk, D), lambda i: (i, 0)),
            pl.BlockSpec((1, D), lambda i: (0, 0)),
            pl.BlockSpec((1, D), lambda i: (0, 0)),
        ],
        out_specs=pl.BlockSpec((blk, D), lambda i: (i, 0)),
        out_shape=jax.ShapeDtypeStruct((N, D), jnp.float32),
    )(x, g, b)


def _post_body(x_ref, h_ref, a_ref, W1_ref, b1_ref, g2_ref, bt2_ref,
               W2_ref, b2_ref, o_ref):
    z = a_ref[...] + h_ref[...]
    z = jnp.dot(z, W1_ref[...], preferred_element_type=jnp.float32) + b1_ref[...]
    mu = jnp.mean(z, axis=-1, keepdims=True)
    var = jnp.mean(jnp.square(z - mu), axis=-1, keepdims=True)
    z = (z - mu) * lax.rsqrt(var + 1e-5) * g2_ref[...] + bt2_ref[...]
    z = jnp.maximum(z, 0.0)
    o_ref[...] = (x_ref[...] + b2_ref[...]
                  + jnp.dot(z, W2_ref[...], preferred_element_type=jnp.float32))


@jax.jit
def _post(x, h, a, W1, b1, g2, bt2, W2, b2):
    blk = 1000
    return pl.pallas_call(
        _post_body,
        grid=(N // blk,),
        in_specs=[
            pl.BlockSpec((blk, D), lambda i: (i, 0)),
            pl.BlockSpec((blk, D), lambda i: (i, 0)),
            pl.BlockSpec((blk, D), lambda i: (i, 0)),
            pl.BlockSpec((D, H), lambda i: (0, 0)),
            pl.BlockSpec((1, H), lambda i: (0, 0)),
            pl.BlockSpec((1, H), lambda i: (0, 0)),
            pl.BlockSpec((1, H), lambda i: (0, 0)),
            pl.BlockSpec((H, D), lambda i: (0, 0)),
            pl.BlockSpec((1, D), lambda i: (0, 0)),
        ],
        out_specs=pl.BlockSpec((blk, D), lambda i: (i, 0)),
        out_shape=jax.ShapeDtypeStruct((N, D), jnp.float32),
    )(x, h, a, W1, b1, g2, bt2, W2, b2)


def kernel(x, edge_index, edge_attr, ln_g, ln_b, t, W1, b1, g2, bt2, W2, b2):
    src = edge_index[0]
    dst = edge_index[1]
    h = _pre(x, ln_g.reshape(1, D), ln_b.reshape(1, D))
    tvec = jnp.broadcast_to(jnp.asarray(t, jnp.float32).reshape(()), (L,))
    agg = _sc_edge(h, src, dst, edge_attr, tvec)[:N]
    return _post(x, h, agg, W1, b1.reshape(1, H), g2.reshape(1, H),
                 bt2.reshape(1, H), W2, b2.reshape(1, D))


# speedup vs baseline: 1.1837x; 1.0025x over previous
import functools

import jax
import jax.numpy as jnp
from jax import lax
from jax.experimental import pallas as pl
from jax.experimental.pallas import tpu as pltpu
from jax.experimental.pallas import tpu_sc as plsc

N = 10000
E = 160000
D = 256
H = 512

NC = 2
NS = 16
L = 16

CHUNK = 2512
N_PAD = 4 * CHUNK
ACC_ROWS = CHUNK + 48
TRASH = CHUNK
EPT = E // NS
SB = 2000


def _sc_edge_body(h_hbm, src_hbm, dst_hbm, ea_hbm, tvec_hbm, agg_hbm,
                  sd, ss, ce_a, ce_b,
                  hb0, hb1, eb0, eb1, ob0, ob1, wbuf, abuf, tvec_v, ix0, ix1,
                  acc, semh0, semh1, seme0, seme1, sems0, sems1):
    c = lax.axis_index("c")
    s = lax.axis_index("s")

    pltpu.sync_copy(tvec_hbm, tvec_v)
    tv = tvec_v[...]

    ebase = s * EPT
    HB = (hb0, hb1)
    EB = (eb0, eb1)
    OB = (ob0, ob1)
    IX = (ix0, ix1)
    SEMH = (semh0, semh1)
    SEME = (seme0, seme1)
    SEMS = (sems0, sems1)

    def _issue(j, b):
        pk = ce_a[pl.ds(j * L, L)]
        isrc = ce_b[pl.ds(j * L, L)]
        pltpu.async_copy(h_hbm.at[isrc], HB[b], SEMH[b])
        pltpu.async_copy(ea_hbm.at[pk & 0x3FFFF], EB[b], SEME[b])

    def _wait(b):
        pltpu.make_async_copy(h_hbm.at[pl.ds(0, L)], HB[b], SEMH[b]).wait()
        pltpu.make_async_copy(ea_hbm.at[pl.ds(0, L)], EB[b], SEME[b]).wait()

    def _drain(b):
        pltpu.make_async_copy(
            h_hbm.at[pl.ds(0, 4 * L), pl.ds(0, 128)], OB[b],
            SEMS[b]).wait()

    def _process(j, b, tvv):
        @pl.when(j >= 2)
        def _():
            _drain(b)

        pk = ce_a[pl.ds(j * L, L)]
        idst4 = (pk >> 18) * 4
        for q in range(4):
            IX[b][q, :] = idst4 + q
        hb, eb, ob = HB[b], EB[b], OB[b]

        def _edge(e, _):
            for g in range(2):
                base = g * 8
                hs = [hb[e, pl.ds((base + i) * L, L)] for i in range(8)]
                es = [eb[e, pl.ds((base + i) * L, L)] for i in range(8)]
                ms = [jnp.maximum(hs[i] + es[i], 0.0) + 1e-7
                      for i in range(8)]
                xs = [jnp.exp(ms[i] * tvv) for i in range(8)]
                ns = [xs[i] * ms[i] for i in range(8)]
                for i in range(8):
                    ob[g * L + e, pl.ds(i * L, L)] = ns[i]
                for i in range(8):
                    ob[(2 + g) * L + e, pl.ds(i * L, L)] = xs[i]
            return 0
        lax.fori_loop(0, L, _edge, 0)

        for q in range(4):
            pltpu.async_copy(ob.at[pl.ds(q * L, L)], acc.at[IX[b].at[q]],
                             SEMS[b], add=True)

    for p in range(2):
        lo = (2 * p + c) * CHUNK

        def _zfill(k, _):
            i = k // 8
            j = (k % 8) * L
            wbuf[i, pl.ds(j, L)] = jnp.zeros((L,), jnp.float32)
            return 0
        lax.fori_loop(0, 32 * 8, _zfill, 0)

        def _zero(i, _):
            pltpu.async_copy(
                wbuf, acc.at[pl.ds((s * (ACC_ROWS // NS) + i * 8) * 4, 32)],
                semh0)
            return 0
        lax.fori_loop(0, ACC_ROWS // NS // 8, _zero, 0)

        def _zdrain(i, _):
            pltpu.make_async_copy(wbuf, acc.at[pl.ds(0, 32)], semh0).wait()
            return 0
        lax.fori_loop(0, ACC_ROWS // NS // 8, _zdrain, 0)
        plsc.subcore_barrier()

        def _strip(cb, _):
            cpd = pltpu.async_copy(
                dst_hbm.at[pl.ds(ebase + cb * SB, SB)], sd, semh0)
            cps = pltpu.async_copy(
                src_hbm.at[pl.ds(ebase + cb * SB, SB)], ss, seme0)
            cpd.wait()
            cps.wait()

            def _compact(i, cnt):
                off = i * L
                d = sd[pl.ds(off, L)]
                dl = d - lo
                m = (dl >= 0) & (dl < CHUNK)
                mi = m.astype(jnp.int32)
                pos = cnt + plsc.cumsum(mi) - 1
                eidv = lax.iota(jnp.int32, L) + (ebase + cb * SB + off)
                plsc.store_scatter(ce_a, [pos], eidv | (dl << 18), mask=m)
                plsc.store_scatter(ce_b, [pos], ss[pl.ds(off, L)], mask=m)
                return cnt + jnp.sum(mi)
            cnt = lax.fori_loop(0, SB // L, _compact, jnp.int32(0))

            ce_a[pl.ds(cnt, L)] = jnp.full((L,), TRASH << 18, jnp.int32)
            ce_b[pl.ds(cnt, L)] = jnp.zeros((L,), jnp.int32)

            nblk = (cnt + L - 1) // L

            @pl.when(nblk > 0)
            def _():
                _issue(0, 0)

            def _pair(i2, _):
                jb = i2 * 2
                for b in range(2):
                    j = jb + b

                    @pl.when(j < nblk)
                    def _():
                        @pl.when(j + 1 < nblk)
                        def _():
                            _issue(j + 1, 1 - b)
                        _wait(b)
                        _process(j, b, tv)
                return 0
            lax.fori_loop(0, (nblk + 1) // 2, _pair, 0)

            for b in range(2):
                @pl.when((nblk >= 1) & ((nblk - 1) % 2 == b))
                def _():
                    _drain(b)

                @pl.when((nblk >= 2) & ((nblk - 2) % 2 == b))
                def _():
                    _drain(b)
            return 0
        lax.fori_loop(0, EPT // SB, _strip, 0)
        plsc.subcore_barrier()

        def _wout(i, _):
            blk = s + NS * i

            @pl.when(blk < CHUNK // 8)
            def _():
                r = blk * 8
                pltpu.sync_copy(acc.at[pl.ds(r * 4, 32)], wbuf)

                @pl.when(i > 0)
                def _():
                    pltpu.make_async_copy(
                        abuf, agg_hbm.at[pl.ds(lo, 8)], sems0).wait()

                def _div(k, _):
                    row = k // (D // L)
                    f = k % (D // L)
                    q = f // 8
                    col = (f % 8) * L
                    nm = wbuf[row * 4 + q, pl.ds(col, L)]
                    dn = wbuf[row * 4 + 2 + q, pl.ds(col, L)]
                    abuf[row, pl.ds(f * L, L)] = nm / (dn + 1e-16)
                    return 0
                lax.fori_loop(0, 8 * D // L, _div, 0)
                pltpu.async_copy(abuf, agg_hbm.at[pl.ds(lo + r, 8)], sems0)
            return 0
        lax.fori_loop(0, (CHUNK // 8 + NS - 1) // NS, _wout, 0)
        pltpu.make_async_copy(abuf, agg_hbm.at[pl.ds(lo, 8)], sems0).wait()
        plsc.subcore_barrier()


@jax.jit
def _sc_edge(h, src, dst, ea, tvec):
    mesh = plsc.VectorSubcoreMesh(core_axis_name="c", subcore_axis_name="s")
    return pl.kernel(
        _sc_edge_body,
        out_type=jax.ShapeDtypeStruct((N_PAD, D), jnp.float32),
        mesh=mesh,
        compiler_params=pltpu.CompilerParams(needs_layout_passes=False),
        scratch_types=[
            pltpu.VMEM((SB,), jnp.int32),
            pltpu.VMEM((SB,), jnp.int32),
            pltpu.VMEM((SB + L,), jnp.int32),
            pltpu.VMEM((SB + L,), jnp.int32),
            pltpu.VMEM((L, D), jnp.float32),
            pltpu.VMEM((L, D), jnp.float32),
            pltpu.VMEM((L, D), jnp.float32),
            pltpu.VMEM((L, D), jnp.float32),
            pltpu.VMEM((4 * L, 128), jnp.float32),
            pltpu.VMEM((4 * L, 128), jnp.float32),
            pltpu.VMEM((32, 128), jnp.float32),
            pltpu.VMEM((8, D), jnp.float32),
            pltpu.VMEM((L,), jnp.float32),
            pltpu.VMEM((4, L), jnp.int32),
            pltpu.VMEM((4, L), jnp.int32),
            pltpu.VMEM_SHARED((ACC_ROWS * 4, 128), jnp.float32),
            pltpu.SemaphoreType.DMA,
            pltpu.SemaphoreType.DMA,
            pltpu.SemaphoreType.DMA,
            pltpu.SemaphoreType.DMA,
            pltpu.SemaphoreType.DMA,
            pltpu.SemaphoreType.DMA,
        ],
    )(h, src, dst, ea, tvec)


def _pre_body(x_ref, g_ref, b_ref, h_ref):
    xv = x_ref[...]
    mu = jnp.mean(xv, axis=-1, keepdims=True)
    var = jnp.mean(jnp.square(xv - mu), axis=-1, keepdims=True)
    hv = (xv - mu) * lax.rsqrt(var + 1e-5) * g_ref[...] + b_ref[...]
    h_ref[...] = jnp.maximum(hv, 0.0)


@jax.jit
def _pre(x, g, b):
    blk = 1000
    return pl.pallas_call(
        _pre_body,
        grid=(N // blk,),
        in_specs=[
            pl.BlockSpec((blk, D), lambda i: (i, 0)),
            pl.BlockSpec((1, D), lambda i: (0, 0)),
            pl.BlockSpec((1, D), lambda i: (0, 0)),
        ],
        out_specs=pl.BlockSpec((blk, D), lambda i: (i, 0)),
        out_shape=jax.ShapeDtypeStruct((N, D), jnp.float32),
    )(x, g, b)


def _post_body(x_ref, h_ref, a_ref, W1_ref, b1_ref, g2_ref, bt2_ref,
               W2_ref, b2_ref, o_ref):
    z = a_ref[...] + h_ref[...]
    z = jnp.dot(z, W1_ref[...], preferred_element_type=jnp.float32) + b1_ref[...]
    mu = jnp.mean(z, axis=-1, keepdims=True)
    var = jnp.mean(jnp.square(z - mu), axis=-1, keepdims=True)
    z = (z - mu) * lax.rsqrt(var + 1e-5) * g2_ref[...] + bt2_ref[...]
    z = jnp.maximum(z, 0.0)
    o_ref[...] = (x_ref[...] + b2_ref[...]
                  + jnp.dot(z, W2_ref[...], preferred_element_type=jnp.float32))


@jax.jit
def _post(x, h, a, W1, b1, g2, bt2, W2, b2):
    blk = 1000
    return pl.pallas_call(
        _post_body,
        grid=(N // blk,),
        in_specs=[
            pl.BlockSpec((blk, D), lambda i: (i, 0)),
            pl.BlockSpec((blk, D), lambda i: (i, 0)),
            pl.BlockSpec((blk, D), lambda i: (i, 0)),
            pl.BlockSpec((D, H), lambda i: (0, 0)),
            pl.BlockSpec((1, H), lambda i: (0, 0)),
            pl.BlockSpec((1, H), lambda i: (0, 0)),
            pl.BlockSpec((1, H), lambda i: (0, 0)),
            pl.BlockSpec((H, D), lambda i: (0, 0)),
            pl.BlockSpec((1, D), lambda i: (0, 0)),
        ],
        out_specs=pl.BlockSpec((blk, D), lambda i: (i, 0)),
        out_shape=jax.ShapeDtypeStruct((N, D), jnp.float32),
    )(x, h, a, W1, b1, g2, bt2, W2, b2)


def kernel(x, edge_index, edge_attr, ln_g, ln_b, t, W1, b1, g2, bt2, W2, b2):
    src = edge_index[0]
    dst = edge_index[1]
    h = _pre(x, ln_g.reshape(1, D), ln_b.reshape(1, D))
    tvec = jnp.broadcast_to(jnp.asarray(t, jnp.float32).reshape(()), (L,))
    agg = _sc_edge(h, src, dst, edge_attr, tvec)[:N]
    return _post(x, h, agg, W1, b1.reshape(1, H), g2.reshape(1, H),
                 bt2.reshape(1, H), W2, b2.reshape(1, D))


# speedup vs baseline: 1.1842x; 1.0004x over previous
import jax
import jax.numpy as jnp
from jax import lax
from jax.experimental import pallas as pl
from jax.experimental.pallas import tpu as pltpu
from jax.experimental.pallas import tpu_sc as plsc

N = 10000
E = 160000
D = 256
H = 512

NS = 16
L = 16

CHUNK = 2512
N_PAD = 4 * CHUNK
ACC_ROWS = CHUNK + 48
TRASH = CHUNK
EPT = E // NS
SB = 2000


def _sc_edge_body(h_hbm, src_hbm, dst_hbm, ea_hbm, tvec_hbm, agg_hbm,
                  sd, ss, ce_a, ce_b,
                  hb0, hb1, eb0, eb1, ob0, ob1, wbuf, abuf, tvec_v, ix0, ix1,
                  acc, semh0, semh1, seme0, seme1, sems0, sems1):
    c = lax.axis_index("c")
    s = lax.axis_index("s")

    pltpu.sync_copy(tvec_hbm, tvec_v)
    tv = tvec_v[...]

    ebase = s * EPT
    HB = (hb0, hb1)
    EB = (eb0, eb1)
    OB = (ob0, ob1)
    IX = (ix0, ix1)
    SEMH = (semh0, semh1)
    SEME = (seme0, seme1)
    SEMS = (sems0, sems1)

    def _issue(j, b):
        pk = ce_a[pl.ds(j * L, L)]
        isrc = ce_b[pl.ds(j * L, L)]
        pltpu.async_copy(h_hbm.at[isrc], HB[b], SEMH[b])
        pltpu.async_copy(ea_hbm.at[pk & 0x3FFFF], EB[b], SEME[b])

    def _wait(b):
        pltpu.make_async_copy(h_hbm.at[pl.ds(0, L)], HB[b], SEMH[b]).wait()
        pltpu.make_async_copy(ea_hbm.at[pl.ds(0, L)], EB[b], SEME[b]).wait()

    def _drain(b):
        pltpu.make_async_copy(
            h_hbm.at[pl.ds(0, 4 * L), pl.ds(0, 128)], OB[b],
            SEMS[b]).wait()

    def _process(j, b, tvv):
        @pl.when(j >= 2)
        def _():
            _drain(b)

        pk = ce_a[pl.ds(j * L, L)]
        idst4 = (pk >> 18) * 4
        for q in range(4):
            IX[b][q, :] = idst4 + q
        hb, eb, ob = HB[b], EB[b], OB[b]

        def _edge(e, _):
            for g in range(2):
                base = g * 8
                hs = [hb[e, pl.ds((base + i) * L, L)] for i in range(8)]
                es = [eb[e, pl.ds((base + i) * L, L)] for i in range(8)]
                ms = [jnp.maximum(hs[i] + es[i], 0.0) + 1e-7
                      for i in range(8)]
                xs = [jnp.exp(ms[i] * tvv) for i in range(8)]
                ns = [xs[i] * ms[i] for i in range(8)]
                for i in range(8):
                    ob[g * L + e, pl.ds(i * L, L)] = ns[i]
                for i in range(8):
                    ob[(2 + g) * L + e, pl.ds(i * L, L)] = xs[i]
            return 0
        lax.fori_loop(0, L, _edge, 0)

        for q in range(4):
            pltpu.async_copy(ob.at[pl.ds(q * L, L)], acc.at[IX[b].at[q]],
                             SEMS[b], add=True)

    for p in range(2):
        lo = (2 * p + c) * CHUNK

        def _zfill(k, _):
            i = k // 8
            j = (k % 8) * L
            wbuf[i, pl.ds(j, L)] = jnp.zeros((L,), jnp.float32)
            return 0
        lax.fori_loop(0, 32 * 8, _zfill, 0)

        def _zero(i, _):
            pltpu.async_copy(
                wbuf, acc.at[pl.ds((s * (ACC_ROWS // NS) + i * 8) * 4, 32)],
                semh0)
            return 0
        lax.fori_loop(0, ACC_ROWS // NS // 8, _zero, 0)

        def _zdrain(i, _):
            pltpu.make_async_copy(wbuf, acc.at[pl.ds(0, 32)], semh0).wait()
            return 0
        lax.fori_loop(0, ACC_ROWS // NS // 8, _zdrain, 0)
        plsc.subcore_barrier()

        def _strip(cb, _):
            cpd = pltpu.async_copy(
                dst_hbm.at[pl.ds(ebase + cb * SB, SB)], sd, semh0)
            cps = pltpu.async_copy(
                src_hbm.at[pl.ds(ebase + cb * SB, SB)], ss, seme0)
            cpd.wait()
            cps.wait()

            def _compact(i, cnt):
                off = i * L
                d = sd[pl.ds(off, L)]
                dl = d - lo
                m = (dl >= 0) & (dl < CHUNK)
                mi = m.astype(jnp.int32)
                pos = cnt + plsc.cumsum(mi) - 1
                eidv = lax.iota(jnp.int32, L) + (ebase + cb * SB + off)
                plsc.store_scatter(ce_a, [pos], eidv | (dl << 18), mask=m)
                plsc.store_scatter(ce_b, [pos], ss[pl.ds(off, L)], mask=m)
                return cnt + jnp.sum(mi)
            cnt = lax.fori_loop(0, SB // L, _compact, jnp.int32(0))

            ce_a[pl.ds(cnt, L)] = jnp.full((L,), TRASH << 18, jnp.int32)
            ce_b[pl.ds(cnt, L)] = jnp.zeros((L,), jnp.int32)

            nblk = (cnt + L - 1) // L

            @pl.when(nblk > 0)
            def _():
                _issue(0, 0)

            def _pair(i2, _):
                jb = i2 * 2
                for b in range(2):
                    j = jb + b

                    @pl.when(j < nblk)
                    def _():
                        @pl.when(j + 1 < nblk)
                        def _():
                            _issue(j + 1, 1 - b)
                        _wait(b)
                        _process(j, b, tv)
                return 0
            lax.fori_loop(0, (nblk + 1) // 2, _pair, 0)

            for b in range(2):
                @pl.when((nblk >= 1) & ((nblk - 1) % 2 == b))
                def _():
                    _drain(b)

                @pl.when((nblk >= 2) & ((nblk - 2) % 2 == b))
                def _():
                    _drain(b)
            return 0
        lax.fori_loop(0, EPT // SB, _strip, 0)
        plsc.subcore_barrier()

        def _wout(i, _):
            blk = s + NS * i

            @pl.when(blk < CHUNK // 8)
            def _():
                r = blk * 8
                pltpu.sync_copy(acc.at[pl.ds(r * 4, 32)], wbuf)

                @pl.when(i > 0)
                def _():
                    pltpu.make_async_copy(
                        abuf, agg_hbm.at[pl.ds(lo, 8)], sems0).wait()

                def _div(k, _):
                    row = k // (D // L)
                    f = k % (D // L)
                    q = f // 8
                    col = (f % 8) * L
                    nm = wbuf[row * 4 + q, pl.ds(col, L)]
                    dn = wbuf[row * 4 + 2 + q, pl.ds(col, L)]
                    abuf[row, pl.ds(f * L, L)] = nm / (dn + 1e-16)
                    return 0
                lax.fori_loop(0, 8 * D // L, _div, 0)
                pltpu.async_copy(abuf, agg_hbm.at[pl.ds(lo + r, 8)], sems0)
            return 0
        lax.fori_loop(0, (CHUNK // 8 + NS - 1) // NS, _wout, 0)
        pltpu.make_async_copy(abuf, agg_hbm.at[pl.ds(lo, 8)], sems0).wait()
        plsc.subcore_barrier()


@jax.jit
def _sc_edge(h, src, dst, ea, tvec):
    mesh = plsc.VectorSubcoreMesh(core_axis_name="c", subcore_axis_name="s")
    return pl.kernel(
        _sc_edge_body,
        out_type=jax.ShapeDtypeStruct((N_PAD, D), jnp.float32),
        mesh=mesh,
        compiler_params=pltpu.CompilerParams(needs_layout_passes=False),
        scratch_types=[
            pltpu.VMEM((SB,), jnp.int32),
            pltpu.VMEM((SB,), jnp.int32),
            pltpu.VMEM((SB + L,), jnp.int32),
            pltpu.VMEM((SB + L,), jnp.int32),
            pltpu.VMEM((L, D), jnp.float32),
            pltpu.VMEM((L, D), jnp.float32),
            pltpu.VMEM((L, D), jnp.float32),
            pltpu.VMEM((L, D), jnp.float32),
            pltpu.VMEM((4 * L, 128), jnp.float32),
            pltpu.VMEM((4 * L, 128), jnp.float32),
            pltpu.VMEM((32, 128), jnp.float32),
            pltpu.VMEM((8, D), jnp.float32),
            pltpu.VMEM((L,), jnp.float32),
            pltpu.VMEM((4, L), jnp.int32),
            pltpu.VMEM((4, L), jnp.int32),
            pltpu.VMEM_SHARED((ACC_ROWS * 4, 128), jnp.float32),
            pltpu.SemaphoreType.DMA,
            pltpu.SemaphoreType.DMA,
            pltpu.SemaphoreType.DMA,
            pltpu.SemaphoreType.DMA,
            pltpu.SemaphoreType.DMA,
            pltpu.SemaphoreType.DMA,
        ],
    )(h, src, dst, ea, tvec)


def _pre_body(x_ref, g_ref, b_ref, h_ref):
    xv = x_ref[...]
    mu = jnp.mean(xv, axis=-1, keepdims=True)
    var = jnp.mean(jnp.square(xv - mu), axis=-1, keepdims=True)
    hv = (xv - mu) * lax.rsqrt(var + 1e-5) * g_ref[...] + b_ref[...]
    h_ref[...] = jnp.maximum(hv, 0.0)


@jax.jit
def _pre(x, g, b):
    blk = 1000
    return pl.pallas_call(
        _pre_body,
        grid=(N // blk,),
        in_specs=[
            pl.BlockSpec((blk, D), lambda i: (i, 0)),
            pl.BlockSpec((1, D), lambda i: (0, 0)),
            pl.BlockSpec((1, D), lambda i: (0, 0)),
        ],
        out_specs=pl.BlockSpec((blk, D), lambda i: (i, 0)),
        out_shape=jax.ShapeDtypeStruct((N, D), jnp.float32),
    )(x, g, b)


def _post_body(x_ref, h_ref, a_ref, W1_ref, b1_ref, g2_ref, bt2_ref,
               W2_ref, b2_ref, o_ref):
    z = a_ref[...] + h_ref[...]
    z = jnp.dot(z, W1_ref[...], preferred_element_type=jnp.float32) + b1_ref[...]
    mu = jnp.mean(z, axis=-1, keepdims=True)
    var = jnp.mean(jnp.square(z - mu), axis=-1, keepdims=True)
    z = (z - mu) * lax.rsqrt(var + 1e-5) * g2_ref[...] + bt2_ref[...]
    z = jnp.maximum(z, 0.0)
    o_ref[...] = (x_ref[...] + b2_ref[...]
                  + jnp.dot(z, W2_ref[...], preferred_element_type=jnp.float32))


@jax.jit
def _post(x, h, a, W1, b1, g2, bt2, W2, b2):
    blk = 1000
    return pl.pallas_call(
        _post_body,
        grid=(N // blk,),
        in_specs=[
            pl.BlockSpec((blk, D), lambda i: (i, 0)),
            pl.BlockSpec((blk, D), lambda i: (i, 0)),
            pl.BlockSpec((blk, D), lambda i: (i, 0)),
            pl.BlockSpec((D, H), lambda i: (0, 0)),
            pl.BlockSpec((1, H), lambda i: (0, 0)),
            pl.BlockSpec((1, H), lambda i: (0, 0)),
            pl.BlockSpec((1, H), lambda i: (0, 0)),
            pl.BlockSpec((H, D), lambda i: (0, 0)),
            pl.BlockSpec((1, D), lambda i: (0, 0)),
        ],
        out_specs=pl.BlockSpec((blk, D), lambda i: (i, 0)),
        out_shape=jax.ShapeDtypeStruct((N, D), jnp.float32),
    )(x, h, a, W1, b1, g2, bt2, W2, b2)


def kernel(x, edge_index, edge_attr, ln_g, ln_b, t, W1, b1, g2, bt2, W2, b2):
    src = edge_index[0]
    dst = edge_index[1]
    h = _pre(x, ln_g.reshape(1, D), ln_b.reshape(1, D))
    tvec = jnp.broadcast_to(jnp.asarray(t, jnp.float32).reshape(()), (L,))
    agg = _sc_edge(h, src, dst, edge_attr, tvec)[:N]
    return _post(x, h, agg, W1, b1.reshape(1, H), g2.reshape(1, H),
                 bt2.reshape(1, H), W2, b2.reshape(1, D))
